# initial kernel scaffold (unmeasured)
import jax
import jax.numpy as jnp
from jax import lax
from jax.experimental import pallas as pl
from jax.experimental.pallas import tpu as pltpu

N_DEV = 4


def kernel(x, w_mat, scale_x, scale_w):
    m_total, k_shard = x.shape
    k_total, n = w_mat.shape
    m_per = m_total // N_DEV

    def body(x_ref, w_ref, sx_ref, sw_ref, out_ref,
             send_buf, recv_buf, send_sems, recv_sems):
        my = lax.axis_index("i")

        barrier = pltpu.get_barrier_semaphore()
        for d in range(1, N_DEV):
            peer = lax.rem(my + d, N_DEV)
            pl.semaphore_signal(barrier, inc=1, device_id=(peer,),
                                device_id_type=pl.DeviceIdType.MESH)
        pl.semaphore_wait(barrier, N_DEV - 1)

        rdmas = []
        for d in range(1, N_DEV):
            dst = lax.rem(my + d, N_DEV)
            send_buf[d - 1] = x_ref[pl.ds(dst * m_per, m_per), :].astype(
                jnp.float8_e4m3fn)
            rdma = pltpu.make_async_remote_copy(
                src_ref=send_buf.at[d - 1],
                dst_ref=recv_buf.at[my],
                send_sem=send_sems.at[d - 1],
                recv_sem=recv_sems.at[my],
                device_id=(dst,),
                device_id_type=pl.DeviceIdType.MESH,
            )
            rdma.start()
            rdmas.append(rdma)

        def wblk(j):
            return w_ref[pl.ds(j * m_per, m_per), :].astype(jnp.bfloat16)

        dot = lambda a, b: lax.dot_general(
            a, b, (((1,), (0,)), ((), ())),
            preferred_element_type=jnp.float32)

        xl = x_ref[pl.ds(my * m_per, m_per), :].astype(jnp.bfloat16)
        out_ref[...] = dot(xl, wblk(my))

        for d in range(1, N_DEV):
            src = lax.rem(my - d + N_DEV, N_DEV)
            recv = pltpu.make_async_remote_copy(
                src_ref=send_buf.at[0],
                dst_ref=recv_buf.at[src],
                send_sem=send_sems.at[0],
                recv_sem=recv_sems.at[src],
                device_id=(my,),
                device_id_type=pl.DeviceIdType.MESH,
            )
            recv.wait_recv()
            xb = recv_buf[src].astype(jnp.bfloat16)
            out_ref[...] += dot(xb, wblk(src))

        for rdma in rdmas:
            rdma.wait_send()

        out_ref[...] = out_ref[...] * (sx_ref[0] * sw_ref[0])

    return pl.pallas_call(
        body,
        out_shape=jax.ShapeDtypeStruct((m_per, n), jnp.float32),
        in_specs=[
            pl.BlockSpec(memory_space=pltpu.VMEM),
            pl.BlockSpec(memory_space=pltpu.VMEM),
            pl.BlockSpec(memory_space=pltpu.SMEM),
            pl.BlockSpec(memory_space=pltpu.SMEM),
        ],
        out_specs=pl.BlockSpec(memory_space=pltpu.VMEM),
        scratch_shapes=[
            pltpu.VMEM((N_DEV - 1, m_per, k_shard), jnp.float8_e4m3fn),
            pltpu.VMEM((N_DEV, m_per, k_shard), jnp.float8_e4m3fn),
            pltpu.SemaphoreType.DMA((N_DEV - 1,)),
            pltpu.SemaphoreType.DMA((N_DEV,)),
        ],
        compiler_params=pltpu.CompilerParams(collective_id=0),
    )(x, w_mat, scale_x, scale_w)


# baseline (device time: 50561 ns/iter reference)
import jax
import jax.numpy as jnp
from jax import lax
from jax.experimental import pallas as pl
from jax.experimental.pallas import tpu as pltpu

N_DEV = 4


def kernel(x, w_mat, scale_x, scale_w):
    m_total, k_shard = x.shape
    k_total, n = w_mat.shape
    m_per = m_total // N_DEV

    def body(x_ref, w_ref, sx_ref, sw_ref, out_ref,
             send_buf, recv_buf, w_vmem, send_sems, recv_sems, w_sems):
        my = lax.axis_index("i")

        jseq = [lax.rem(my - d + N_DEV, N_DEV) for d in range(N_DEV)]

        def w_copy(k):
            return pltpu.make_async_copy(
                w_ref.at[pl.ds(jseq[k] * m_per, m_per), :],
                w_vmem.at[k % 2],
                w_sems.at[k % 2],
            )

        w_copy(0).start()
        w_copy(1).start()

        barrier = pltpu.get_barrier_semaphore()
        for d in range(1, N_DEV):
            peer = lax.rem(my + d, N_DEV)
            pl.semaphore_signal(barrier, inc=1, device_id=(peer,),
                                device_id_type=pl.DeviceIdType.MESH)
        pl.semaphore_wait(barrier, N_DEV - 1)

        rdmas = []
        for d in range(1, N_DEV):
            dst = lax.rem(my + d, N_DEV)
            send_buf[d - 1] = x_ref[pl.ds(dst * m_per, m_per), :].astype(
                jnp.float8_e4m3fn)
            rdma = pltpu.make_async_remote_copy(
                src_ref=send_buf.at[d - 1],
                dst_ref=recv_buf.at[my],
                send_sem=send_sems.at[d - 1],
                recv_sem=recv_sems.at[my],
                device_id=(dst,),
                device_id_type=pl.DeviceIdType.MESH,
            )
            rdma.start()
            rdmas.append(rdma)

        dot = lambda a, b: lax.dot_general(
            a, b, (((1,), (0,)), ((), ())),
            preferred_element_type=jnp.float32)

        xl = x_ref[pl.ds(my * m_per, m_per), :].astype(jnp.bfloat16)
        w_copy(0).wait()
        out_ref[...] = dot(xl, w_vmem[0].astype(jnp.bfloat16))
        w_copy(2).start()

        for d in range(1, N_DEV):
            src = jseq[d]
            recv = pltpu.make_async_remote_copy(
                src_ref=send_buf.at[0],
                dst_ref=recv_buf.at[src],
                send_sem=send_sems.at[0],
                recv_sem=recv_sems.at[src],
                device_id=(my,),
                device_id_type=pl.DeviceIdType.MESH,
            )
            recv.wait_recv()
            w_copy(d).wait()
            xb = recv_buf[src].astype(jnp.bfloat16)
            out_ref[...] += dot(xb, w_vmem[d % 2].astype(jnp.bfloat16))
            if d + 2 < N_DEV:
                w_copy(d + 2).start()

        for rdma in rdmas:
            rdma.wait_send()

        out_ref[...] = out_ref[...] * (sx_ref[0] * sw_ref[0])

    return pl.pallas_call(
        body,
        out_shape=jax.ShapeDtypeStruct((m_per, n), jnp.float32),
        in_specs=[
            pl.BlockSpec(memory_space=pltpu.VMEM),
            pl.BlockSpec(memory_space=pl.ANY),
            pl.BlockSpec(memory_space=pltpu.SMEM),
            pl.BlockSpec(memory_space=pltpu.SMEM),
        ],
        out_specs=pl.BlockSpec(memory_space=pltpu.VMEM),
        scratch_shapes=[
            pltpu.VMEM((N_DEV - 1, m_per, k_shard), jnp.float8_e4m3fn),
            pltpu.VMEM((N_DEV, m_per, k_shard), jnp.float8_e4m3fn),
            pltpu.VMEM((2, m_per, n), jnp.float32),
            pltpu.SemaphoreType.DMA((N_DEV - 1,)),
            pltpu.SemaphoreType.DMA((N_DEV,)),
            pltpu.SemaphoreType.DMA((2,)),
        ],
        compiler_params=pltpu.CompilerParams(
            collective_id=0, vmem_limit_bytes=100 * 1024 * 1024),
    )(x, w_mat, scale_x, scale_w)


# device time: 47398 ns/iter; 1.0667x vs baseline; 1.0667x over previous
import jax
import jax.numpy as jnp
from jax import lax
from jax.experimental import pallas as pl
from jax.experimental.pallas import tpu as pltpu

N_DEV = 4


def kernel(x, w_mat, scale_x, scale_w):
    m_total, k_shard = x.shape
    k_total, n = w_mat.shape
    m_per = m_total // N_DEV

    def body(x_ref, w_ref, sx_ref, sw_ref, out_ref,
             send_buf, recv_buf, w_vmem, send_sems, recv_sems, w_sems):
        my = lax.axis_index("i")

        jseq = [lax.rem(my - d + N_DEV, N_DEV) for d in range(N_DEV)]

        def w_copy(k):
            return pltpu.make_async_copy(
                w_ref.at[pl.ds(jseq[k] * m_per, m_per), :],
                w_vmem.at[k % 2],
                w_sems.at[k % 2],
            )

        w_copy(0).start()
        w_copy(1).start()

        barrier = pltpu.get_barrier_semaphore()
        for d in range(1, N_DEV):
            peer = lax.rem(my + d, N_DEV)
            pl.semaphore_signal(barrier, inc=1, device_id=(peer,),
                                device_id_type=pl.DeviceIdType.MESH)
        pl.semaphore_wait(barrier, N_DEV - 1)

        rdmas = []
        for d in range(1, N_DEV):
            dst = lax.rem(my + d, N_DEV)
            send_buf[d - 1] = x_ref[pl.ds(dst * m_per, m_per), :].astype(
                jnp.float8_e4m3fn)
            rdma = pltpu.make_async_remote_copy(
                src_ref=send_buf.at[d - 1],
                dst_ref=recv_buf.at[my],
                send_sem=send_sems.at[d - 1],
                recv_sem=recv_sems.at[my],
                device_id=(dst,),
                device_id_type=pl.DeviceIdType.MESH,
            )
            rdma.start()
            rdmas.append(rdma)
        recv_buf[my] = x_ref[pl.ds(my * m_per, m_per), :].astype(
            jnp.float8_e4m3fn)

        dot = lambda a, b: lax.dot_general(
            a, b, (((1,), (0,)), ((), ())),
            preferred_element_type=jnp.float32)

        w_copy(0).wait()
        out_ref[...] = dot(recv_buf[my], w_vmem[0].astype(jnp.float8_e5m2))
        w_copy(2).start()

        for d in range(1, N_DEV):
            src = jseq[d]
            recv = pltpu.make_async_remote_copy(
                src_ref=send_buf.at[0],
                dst_ref=recv_buf.at[src],
                send_sem=send_sems.at[0],
                recv_sem=recv_sems.at[src],
                device_id=(my,),
                device_id_type=pl.DeviceIdType.MESH,
            )
            recv.wait_recv()
            w_copy(d).wait()
            out_ref[...] += dot(recv_buf[src],
                                w_vmem[d % 2].astype(jnp.float8_e5m2))
            if d + 2 < N_DEV:
                w_copy(d + 2).start()

        for rdma in rdmas:
            rdma.wait_send()

        out_ref[...] = out_ref[...] * (sx_ref[0] * sw_ref[0])

    return pl.pallas_call(
        body,
        out_shape=jax.ShapeDtypeStruct((m_per, n), jnp.float32),
        in_specs=[
            pl.BlockSpec(memory_space=pltpu.VMEM),
            pl.BlockSpec(memory_space=pl.ANY),
            pl.BlockSpec(memory_space=pltpu.SMEM),
            pl.BlockSpec(memory_space=pltpu.SMEM),
        ],
        out_specs=pl.BlockSpec(memory_space=pltpu.VMEM),
        scratch_shapes=[
            pltpu.VMEM((N_DEV - 1, m_per, k_shard), jnp.float8_e4m3fn),
            pltpu.VMEM((N_DEV, m_per, k_shard), jnp.float8_e4m3fn),
            pltpu.VMEM((2, m_per, n), jnp.float32),
            pltpu.SemaphoreType.DMA((N_DEV - 1,)),
            pltpu.SemaphoreType.DMA((N_DEV,)),
            pltpu.SemaphoreType.DMA((2,)),
        ],
        compiler_params=pltpu.CompilerParams(
            collective_id=0, vmem_limit_bytes=100 * 1024 * 1024),
    )(x, w_mat, scale_x, scale_w)


# device time: 30536 ns/iter; 1.6558x vs baseline; 1.5522x over previous
import os

import jax
import jax.numpy as jnp
from jax import lax
from jax.experimental import pallas as pl
from jax.experimental.pallas import tpu as pltpu

N_DEV = 4
_ABLATE = os.environ.get("ABLATE", "")


def kernel(x, w_mat, scale_x, scale_w):
    m_total, k_shard = x.shape
    k_total, n = w_mat.shape
    m_per = m_total // N_DEV

    def body(x_ref, w_ref, sx_ref, sw_ref, out_ref,
             send_buf, recv_buf, w_vmem, send_sems, recv_sems, w_sems):
        my = lax.axis_index("i")

        jseq = [lax.rem(my - d + N_DEV, N_DEV) for d in range(N_DEV)]

        def w_copy(k):
            return pltpu.make_async_copy(
                w_ref.at[pl.ds(jseq[k] * m_per, m_per), :],
                w_vmem.at[k % 2],
                w_sems.at[k % 2],
            )

        w_copy(0).start()
        w_copy(1).start()

        with jax.named_scope("barrier"):
            barrier = pltpu.get_barrier_semaphore()
            for d in range(1, N_DEV):
                peer = lax.rem(my + d, N_DEV)
                pl.semaphore_signal(barrier, inc=1, device_id=(peer,),
                                    device_id_type=pl.DeviceIdType.MESH)
            pl.semaphore_wait(barrier, N_DEV - 1)

        rdmas = []
        with jax.named_scope("stage_send"):
            for d in range(1, N_DEV):
                dst = lax.rem(my + d, N_DEV)
                send_buf[d - 1] = x_ref[pl.ds(dst * m_per, m_per), :].astype(
                    jnp.float8_e4m3fn)
                rdma = pltpu.make_async_remote_copy(
                    src_ref=send_buf.at[d - 1],
                    dst_ref=recv_buf.at[my],
                    send_sem=send_sems.at[d - 1],
                    recv_sem=recv_sems.at[my],
                    device_id=(dst,),
                    device_id_type=pl.DeviceIdType.MESH,
                )
                if _ABLATE != "nocomm":
                    rdma.start()
                    rdmas.append(rdma)
            recv_buf[my] = x_ref[pl.ds(my * m_per, m_per), :].astype(
                jnp.float8_e4m3fn)

        dot = lambda a, b: lax.dot_general(
            a, b, (((1,), (0,)), ((), ())),
            preferred_element_type=jnp.float32)

        with jax.named_scope("local_dot"):
            w_copy(0).wait()
            out_ref[...] = dot(recv_buf[my],
                               w_vmem[0].astype(jnp.float8_e5m2))
            w_copy(2).start()

        for d in range(1, N_DEV):
            src = jseq[d]
            recv = pltpu.make_async_remote_copy(
                src_ref=send_buf.at[0],
                dst_ref=recv_buf.at[src],
                send_sem=send_sems.at[0],
                recv_sem=recv_sems.at[src],
                device_id=(my,),
                device_id_type=pl.DeviceIdType.MESH,
            )
            with jax.named_scope(f"wait_recv#hop={d}"):
                if _ABLATE != "nocomm":
                    recv.wait_recv()
                w_copy(d).wait()
            with jax.named_scope(f"dot#hop={d}"):
                if _ABLATE != "nocompute":
                    out_ref[...] += dot(recv_buf[src],
                                        w_vmem[d % 2].astype(jnp.float8_e5m2))
                if d + 2 < N_DEV:
                    w_copy(d + 2).start()

        with jax.named_scope("tail"):
            for rdma in rdmas:
                rdma.wait_send()

            out_ref[...] = out_ref[...] * (sx_ref[0] * sw_ref[0])

    return pl.pallas_call(
        body,
        out_shape=jax.ShapeDtypeStruct((m_per, n), jnp.float32),
        in_specs=[
            pl.BlockSpec(memory_space=pltpu.VMEM),
            pl.BlockSpec(memory_space=pl.ANY),
            pl.BlockSpec(memory_space=pltpu.SMEM),
            pl.BlockSpec(memory_space=pltpu.SMEM),
        ],
        out_specs=pl.BlockSpec(memory_space=pltpu.VMEM),
        scratch_shapes=[
            pltpu.VMEM((N_DEV - 1, m_per, k_shard), jnp.float8_e4m3fn),
            pltpu.VMEM((N_DEV, m_per, k_shard), jnp.float8_e4m3fn),
            pltpu.VMEM((2, m_per, n), jnp.float32),
            pltpu.SemaphoreType.DMA((N_DEV - 1,)),
            pltpu.SemaphoreType.DMA((N_DEV,)),
            pltpu.SemaphoreType.DMA((2,)),
        ],
        compiler_params=pltpu.CompilerParams(
            collective_id=0, vmem_limit_bytes=100 * 1024 * 1024),
    )(x, w_mat, scale_x, scale_w)
